# Initial kernel scaffold; baseline (speedup 1.0000x reference)
import jax
import jax.numpy as jnp
from jax.experimental import pallas as pl


def kernel(feat, edge_index, W0, b0, W1, b1, Wr):
    raise NotImplementedError("write your pallas kernel here")


# speedup vs baseline: 5.1032x; 5.1032x over previous
import functools

import jax
import jax.numpy as jnp
from jax import lax
from jax.experimental import pallas as pl
from jax.experimental.pallas import tpu as pltpu
from jax.experimental.pallas import tpu_sc as plsc

NC = 2
NS = 16
LN = 16
BM = 512


def _rsqrt_newton(x):
    bits = plsc.bitcast(x, jnp.int32)
    bits = 0x5F3759DF - lax.shift_right_arithmetic(bits, 1)
    y = plsc.bitcast(bits, jnp.float32)
    for _ in range(3):
        y = y * (1.5 - 0.5 * x * y * y)
    return y


def _make_deg_kernel(NP, RA):
    SL = NP // NS
    mesh = plsc.VectorSubcoreMesh(core_axis_name="c", subcore_axis_name="s",
                                  num_cores=NC, num_subcores=NS)

    @functools.partial(
        pl.kernel, mesh=mesh,
        out_type=jax.ShapeDtypeStruct((NC, 2, NP), jnp.float32),
        scratch_types=[
            pltpu.VMEM((RA, 128), jnp.int32),
            pltpu.VMEM((RA, 128), jnp.int32),
            pltpu.VMEM((NP,), jnp.float32),
            pltpu.VMEM((NP,), jnp.float32),
            pltpu.VMEM((2, SL), jnp.float32),
            pltpu.VMEM((2, SL), jnp.float32),
            pltpu.VMEM_SHARED((NS, 2, NP), jnp.float32),
        ],
        compiler_params=pltpu.CompilerParams(needs_layout_passes=False),
    )
    def deg_kernel(src_hbm, dst_hbm, out_hbm, sidx, didx, dego, degi,
                   accv, tmpv, stage):
        cid = lax.axis_index("c")
        sid = lax.axis_index("s")
        pltpu.sync_copy(src_hbm.at[pl.ds(sid * RA, RA)], sidx)
        pltpu.sync_copy(dst_hbm.at[pl.ds(sid * RA, RA)], didx)
        zeros16 = jnp.zeros((LN,), jnp.float32)

        def zbody(i, _):
            dego[pl.ds(i * LN, LN)] = zeros16
            degi[pl.ds(i * LN, LN)] = zeros16
            return 0
        lax.fori_loop(0, NP // LN, zbody, 0)

        ones16 = jnp.full((LN,), 1.0, jnp.float32)

        def ebody(j, _):
            for g in range(128 // LN):
                si = sidx[j, pl.ds(g * LN, LN)]
                plsc.addupdate_scatter(dego, [si], ones16)
                di = didx[j, pl.ds(g * LN, LN)]
                plsc.addupdate_scatter(degi, [di], ones16)
            return 0
        lax.fori_loop(0, RA, ebody, 0)

        pltpu.sync_copy(dego, stage.at[sid, 0])
        pltpu.sync_copy(degi, stage.at[sid, 1])
        plsc.subcore_barrier()

        base = sid * SL
        pltpu.sync_copy(stage.at[0, :, pl.ds(base, SL)], accv)
        for p in range(1, NS):
            pltpu.sync_copy(stage.at[p, :, pl.ds(base, SL)], tmpv)

            def abody(i, _):
                for r in range(2):
                    s = pl.ds(i * LN, LN)
                    accv[r, s] = accv[r, s] + tmpv[r, s]
                return 0
            lax.fori_loop(0, SL // LN, abody, 0)

        def nbody(i, _):
            for r in range(2):
                s = pl.ds(i * LN, LN)
                accv[r, s] = _rsqrt_newton(jnp.maximum(accv[r, s], 1.0))
            return 0
        lax.fori_loop(0, SL // LN, nbody, 0)
        pltpu.sync_copy(accv, out_hbm.at[cid, :, pl.ds(base, SL)])

    return deg_kernel


def _make_agg_kernel(NP, KJ):
    SL = NP // NS
    mesh = plsc.VectorSubcoreMesh(core_axis_name="c", subcore_axis_name="s",
                                  num_cores=NC, num_subcores=NS)

    @functools.partial(
        pl.kernel, mesh=mesh,
        out_type=jax.ShapeDtypeStruct((NC, NP, 128), jnp.float32),
        scratch_types=[
            pltpu.VMEM((KJ, 128), jnp.int32),
            pltpu.VMEM((KJ, 128), jnp.int32),
            pltpu.VMEM((128, 128), jnp.float32),
            pltpu.VMEM_SHARED((NP, 128), jnp.float32),
            pltpu.SemaphoreType.DMA,
        ],
    )
    def agg_kernel(hs_hbm, src_hbm, dst_hbm, zz_hbm, out_hbm,
                   sidx, didx, rows, acc, sem):
        cid = lax.axis_index("c")
        sid = lax.axis_index("s")
        wid = sid * NC + cid
        pltpu.sync_copy(src_hbm.at[pl.ds(wid * KJ, KJ)], sidx)
        pltpu.sync_copy(dst_hbm.at[pl.ds(wid * KJ, KJ)], didx)
        pltpu.sync_copy(zz_hbm, acc.at[pl.ds(sid * SL, SL)])
        plsc.subcore_barrier()

        def body(j, _):
            pltpu.async_copy(hs_hbm.at[sidx.at[j]], rows, sem).wait()
            pltpu.sync_copy(rows, acc.at[didx.at[j]], add=True)
            return 0
        lax.fori_loop(0, KJ, body, 0)

        plsc.subcore_barrier()
        pltpu.sync_copy(acc.at[pl.ds(sid * SL, SL)],
                        out_hbm.at[cid].at[pl.ds(sid * SL, SL)])

    return agg_kernel


def _mm_scale_body(x_ref, w_ref, ns_ref, o_ref):
    o_ref[...] = jnp.dot(x_ref[...], w_ref[...],
                         preferred_element_type=jnp.float32) * ns_ref[...]


def _post_mm_body(a0_ref, a1_ref, nd_ref, b_ref, w_ref, ns_ref, o_ref):
    x = (a0_ref[...] + a1_ref[...]) * nd_ref[...] + b_ref[...]
    x = jnp.maximum(x, 0.0)
    o_ref[...] = jnp.dot(x, w_ref[...],
                         preferred_element_type=jnp.float32) * ns_ref[...]


def _make_readout_body(NN, NB, C):
    def readout_body(a0_ref, a1_ref, nd_ref, b_ref, wr_ref, o_ref, acc_ref):
        i = pl.program_id(0)
        x = (a0_ref[...] + a1_ref[...]) * nd_ref[...] + b_ref[...]
        x = jnp.maximum(x, 0.0)
        rowid = i * BM + lax.broadcasted_iota(jnp.int32, (BM, 128), 0)
        x = jnp.where(rowid < NN, x, 0.0)
        s = jnp.sum(x, axis=0, keepdims=True)

        @pl.when(i == 0)
        def _():
            acc_ref[...] = s

        @pl.when(i > 0)
        def _():
            acc_ref[...] = acc_ref[...] + s

        @pl.when(i == NB - 1)
        def _():
            o_ref[...] = jnp.dot(acc_ref[...] / NN, wr_ref[...],
                                 preferred_element_type=jnp.float32)
    return readout_body


def kernel(feat, edge_index, W0, b0, W1, b1, Wr):
    NN, D = feat.shape
    E = edge_index.shape[1]
    H = W0.shape[1]
    C = Wr.shape[1]

    NP = -(-(NN + 1) // BM) * BM
    EP = -(-E // (NC * NS * 128 * 8)) * (NC * NS * 128 * 8)
    KJ = EP // (NC * NS) // 128
    RA = EP // NS // 128
    NB = NP // BM
    SL = NP // NS

    src = edge_index[0]
    dst = edge_index[1]
    padi = jnp.full((EP - E,), NN, jnp.int32)
    src2d = jnp.concatenate([src, padi]).reshape(EP // 128, 128)
    dst2d = jnp.concatenate([dst, padi]).reshape(EP // 128, 128)
    feat_p = jnp.pad(feat, ((0, NP - NN), (0, 0)))
    zz = jnp.zeros((SL, 128), jnp.float32)

    norms = _make_deg_kernel(NP, RA)(src2d, dst2d)
    ns = norms[0, 0].reshape(NP, 1)
    nd = norms[0, 1].reshape(NP, 1)

    row_spec = pl.BlockSpec((BM, 128), lambda i: (i, 0))
    col_spec = pl.BlockSpec((BM, 1), lambda i: (i, 0))
    w_spec = pl.BlockSpec((D, H), lambda i: (0, 0))
    b_spec = pl.BlockSpec((1, 128), lambda i: (0, 0))

    hs1 = pl.pallas_call(
        _mm_scale_body,
        grid=(NB,),
        in_specs=[row_spec, w_spec, col_spec],
        out_specs=row_spec,
        out_shape=jax.ShapeDtypeStruct((NP, H), jnp.float32),
    )(feat_p, W0, ns)

    agg = _make_agg_kernel(NP, KJ)
    p1 = agg(hs1, src2d, dst2d, zz)

    hs2 = pl.pallas_call(
        _post_mm_body,
        grid=(NB,),
        in_specs=[row_spec, row_spec, col_spec, b_spec, w_spec, col_spec],
        out_specs=row_spec,
        out_shape=jax.ShapeDtypeStruct((NP, H), jnp.float32),
    )(p1[0], p1[1], nd, b0.reshape(1, H), W1, ns)

    p2 = agg(hs2, src2d, dst2d, zz)

    out = pl.pallas_call(
        _make_readout_body(NN, NB, C),
        grid=(NB,),
        in_specs=[row_spec, row_spec, col_spec, b_spec,
                  pl.BlockSpec((H, C), lambda i: (0, 0))],
        out_specs=pl.BlockSpec((1, C), lambda i: (0, 0)),
        out_shape=jax.ShapeDtypeStruct((1, C), jnp.float32),
        scratch_shapes=[pltpu.VMEM((1, 128), jnp.float32)],
    )(p2[0], p2[1], nd, b1.reshape(1, H), Wr)

    return out


# speedup vs baseline: 5.2742x; 1.0335x over previous
import functools

import jax
import jax.numpy as jnp
from jax import lax
from jax.experimental import pallas as pl
from jax.experimental.pallas import tpu as pltpu
from jax.experimental.pallas import tpu_sc as plsc

NC = 2
NS = 16
LN = 16
BM = 512
SUP = 8


def _rsqrt_newton(x):
    bits = plsc.bitcast(x, jnp.int32)
    bits = 0x5F3759DF - lax.shift_right_arithmetic(bits, 1)
    y = plsc.bitcast(bits, jnp.float32)
    for _ in range(3):
        y = y * (1.5 - 0.5 * x * y * y)
    return y


def _make_deg_kernel(NP, RA):
    SL = NP // NS
    mesh = plsc.VectorSubcoreMesh(core_axis_name="c", subcore_axis_name="s",
                                  num_cores=NC, num_subcores=NS)

    @functools.partial(
        pl.kernel, mesh=mesh,
        out_type=jax.ShapeDtypeStruct((NC, 2, NP), jnp.float32),
        scratch_types=[
            pltpu.VMEM((RA, 128), jnp.int32),
            pltpu.VMEM((RA, 128), jnp.int32),
            pltpu.VMEM((NP,), jnp.float32),
            pltpu.VMEM((NP,), jnp.float32),
            pltpu.VMEM((2, SL), jnp.float32),
            pltpu.VMEM((2, SL), jnp.float32),
            pltpu.VMEM_SHARED((NS, 2, NP), jnp.float32),
        ],
        compiler_params=pltpu.CompilerParams(needs_layout_passes=False),
    )
    def deg_kernel(src_hbm, dst_hbm, out_hbm, sidx, didx, dego, degi,
                   accv, tmpv, stage):
        cid = lax.axis_index("c")
        sid = lax.axis_index("s")
        pltpu.sync_copy(src_hbm.at[pl.ds(sid * RA, RA)], sidx)
        pltpu.sync_copy(dst_hbm.at[pl.ds(sid * RA, RA)], didx)
        zeros16 = jnp.zeros((LN,), jnp.float32)

        def zbody(i, _):
            dego[pl.ds(i * LN, LN)] = zeros16
            degi[pl.ds(i * LN, LN)] = zeros16
            return 0
        lax.fori_loop(0, NP // LN, zbody, 0)

        ones16 = jnp.full((LN,), 1.0, jnp.float32)

        def ebody(j, _):
            for g in range(128 // LN):
                si = sidx[j, pl.ds(g * LN, LN)]
                plsc.addupdate_scatter(dego, [si], ones16)
                di = didx[j, pl.ds(g * LN, LN)]
                plsc.addupdate_scatter(degi, [di], ones16)
            return 0
        lax.fori_loop(0, RA, ebody, 0)

        pltpu.sync_copy(dego, stage.at[sid, 0])
        pltpu.sync_copy(degi, stage.at[sid, 1])
        plsc.subcore_barrier()

        base = sid * SL
        pltpu.sync_copy(stage.at[0, :, pl.ds(base, SL)], accv)
        for p in range(1, NS):
            pltpu.sync_copy(stage.at[p, :, pl.ds(base, SL)], tmpv)

            def abody(i, _):
                for r in range(2):
                    s = pl.ds(i * LN, LN)
                    accv[r, s] = accv[r, s] + tmpv[r, s]
                return 0
            lax.fori_loop(0, SL // LN, abody, 0)

        def nbody(i, _):
            for r in range(2):
                s = pl.ds(i * LN, LN)
                accv[r, s] = _rsqrt_newton(jnp.maximum(accv[r, s], 1.0))
            return 0
        lax.fori_loop(0, SL // LN, nbody, 0)
        pltpu.sync_copy(accv, out_hbm.at[cid, :, pl.ds(base, SL)])

    return deg_kernel


def _make_agg_kernel(NP, KJ):
    SL = NP // NS
    NSS = KJ // SUP
    mesh = plsc.VectorSubcoreMesh(core_axis_name="c", subcore_axis_name="s",
                                  num_cores=NC, num_subcores=NS)

    @functools.partial(
        pl.kernel, mesh=mesh,
        out_type=jax.ShapeDtypeStruct((NC, NP, 128), jnp.float32),
        scratch_types=[
            pltpu.VMEM((KJ, 128), jnp.int32),
            pltpu.VMEM((2, SUP, 128), jnp.int32),
            pltpu.VMEM((128, 128), jnp.float32),
            pltpu.VMEM((128, 128), jnp.float32),
            pltpu.VMEM_SHARED((NP, 128), jnp.float32),
        ] + [pltpu.SemaphoreType.DMA] * 5,
    )
    def agg_kernel(hs_hbm, src_hbm, dst_hbm, zz_hbm, out_hbm,
                   sidx, dbuf, r0, r1, acc, gs0, gs1, ss0, ss1, dsem):
        rows = (r0, r1)
        gs = (gs0, gs1)
        ss = (ss0, ss1)
        cid = lax.axis_index("c")
        sid = lax.axis_index("s")
        wid = sid * NC + cid
        base = wid * KJ
        pltpu.sync_copy(src_hbm.at[pl.ds(base, KJ)], sidx)
        pltpu.sync_copy(dst_hbm.at[pl.ds(base, SUP)], dbuf.at[0])
        pltpu.sync_copy(zz_hbm, acc.at[pl.ds(sid * SL, SL)])
        plsc.subcore_barrier()

        pltpu.async_copy(hs_hbm.at[sidx.at[0]], rows[0], gs[0])
        pltpu.async_copy(hs_hbm.at[sidx.at[1]], rows[1], gs[1])

        def body(g, _):
            gmod = g % 2

            @pl.when(g > 0)
            def _():
                pltpu.make_async_copy(dst_hbm.at[pl.ds(base, SUP)],
                                      dbuf.at[0], dsem).wait()

            for cc in range(SUP):
                b = cc % 2
                c = g * SUP + cc
                pltpu.make_async_copy(hs_hbm.at[sidx.at[c]],
                                      rows[b], gs[b]).wait()
                pltpu.async_copy(rows[b], acc.at[dbuf.at[gmod, cc]],
                                 ss[b], add=True)
                if cc == 1:
                    @pl.when(g < NSS - 1)
                    def _():
                        off = pl.multiple_of(base + (g + 1) * SUP, SUP)
                        pltpu.async_copy(
                            dst_hbm.at[pl.ds(off, SUP)],
                            dbuf.at[(g + 1) % 2], dsem)
                if cc == 0:
                    @pl.when(g > 0)
                    def _():
                        pltpu.make_async_copy(
                            rows[1], acc.at[dbuf.at[0, 0]], ss[1]).wait()
                        pltpu.async_copy(hs_hbm.at[sidx.at[c + 1]],
                                         rows[1], gs[1])
                else:
                    b1 = (cc - 1) % 2
                    pltpu.make_async_copy(
                        rows[b1], acc.at[dbuf.at[0, 0]], ss[b1]).wait()

                    @pl.when(c < KJ - 1)
                    def _(b1=b1, c=c):
                        pltpu.async_copy(hs_hbm.at[sidx.at[c + 1]],
                                         rows[b1], gs[b1])
            return 0
        lax.fori_loop(0, NSS, body, 0)

        pltpu.make_async_copy(rows[1], acc.at[dbuf.at[0, 0]], ss[1]).wait()
        plsc.subcore_barrier()
        pltpu.sync_copy(acc.at[pl.ds(sid * SL, SL)],
                        out_hbm.at[cid].at[pl.ds(sid * SL, SL)])

    return agg_kernel


def _mm_scale_body(x_ref, w_ref, ns_ref, o_ref):
    o_ref[...] = jnp.dot(x_ref[...], w_ref[...],
                         preferred_element_type=jnp.float32) * ns_ref[...]


def _post_mm_body(a_ref, nd_ref, b_ref, w_ref, ns_ref, o_ref):
    x = (a_ref[0] + a_ref[1]) * nd_ref[...] + b_ref[...]
    x = jnp.maximum(x, 0.0)
    o_ref[...] = jnp.dot(x, w_ref[...],
                         preferred_element_type=jnp.float32) * ns_ref[...]


def _make_readout_body(NN, NB, C):
    def readout_body(a_ref, nd_ref, b_ref, wr_ref, o_ref, acc_ref):
        i = pl.program_id(0)
        x = (a_ref[0] + a_ref[1]) * nd_ref[...] + b_ref[...]
        x = jnp.maximum(x, 0.0)
        rowid = i * BM + lax.broadcasted_iota(jnp.int32, (BM, 128), 0)
        x = jnp.where(rowid < NN, x, 0.0)
        s = jnp.sum(x, axis=0, keepdims=True)

        @pl.when(i == 0)
        def _():
            acc_ref[...] = s

        @pl.when(i > 0)
        def _():
            acc_ref[...] = acc_ref[...] + s

        @pl.when(i == NB - 1)
        def _():
            o_ref[...] = jnp.dot(acc_ref[...] / NN, wr_ref[...],
                                 preferred_element_type=jnp.float32)
    return readout_body


def kernel(feat, edge_index, W0, b0, W1, b1, Wr):
    NN, D = feat.shape
    E = edge_index.shape[1]
    H = W0.shape[1]
    C = Wr.shape[1]

    NP = -(-(NN + 1) // BM) * BM
    EP = -(-E // (NC * NS * 128 * 8)) * (NC * NS * 128 * 8)
    KJ = EP // (NC * NS) // 128
    RA = EP // NS // 128
    NB = NP // BM
    SL = NP // NS

    src = edge_index[0]
    dst = edge_index[1]
    padi = jnp.full((EP - E,), NN, jnp.int32)
    src2d = jnp.concatenate([src, padi]).reshape(EP // 128, 128)
    dst2d = jnp.concatenate([dst, padi]).reshape(EP // 128, 128)
    feat_p = jnp.pad(feat, ((0, NP - NN), (0, 0)))
    zz = jnp.zeros((SL, 128), jnp.float32)

    norms = _make_deg_kernel(NP, RA)(src2d, dst2d)
    ns = norms[0, 0].reshape(NP, 1)
    nd = norms[0, 1].reshape(NP, 1)

    row_spec = pl.BlockSpec((BM, 128), lambda i: (i, 0))
    col_spec = pl.BlockSpec((BM, 1), lambda i: (i, 0))
    w_spec = pl.BlockSpec((D, H), lambda i: (0, 0))
    b_spec = pl.BlockSpec((1, 128), lambda i: (0, 0))
    pair_spec = pl.BlockSpec((NC, BM, 128), lambda i: (0, i, 0))

    hs1 = pl.pallas_call(
        _mm_scale_body,
        grid=(NB,),
        in_specs=[row_spec, w_spec, col_spec],
        out_specs=row_spec,
        out_shape=jax.ShapeDtypeStruct((NP, H), jnp.float32),
    )(feat_p, W0, ns)

    agg = _make_agg_kernel(NP, KJ)
    p1 = agg(hs1, src2d, dst2d, zz)

    hs2 = pl.pallas_call(
        _post_mm_body,
        grid=(NB,),
        in_specs=[pair_spec, col_spec, b_spec, w_spec, col_spec],
        out_specs=row_spec,
        out_shape=jax.ShapeDtypeStruct((NP, H), jnp.float32),
    )(p1, nd, b0.reshape(1, H), W1, ns)

    p2 = agg(hs2, src2d, dst2d, zz)

    out = pl.pallas_call(
        _make_readout_body(NN, NB, C),
        grid=(NB,),
        in_specs=[pair_spec, col_spec, b_spec,
                  pl.BlockSpec((H, C), lambda i: (0, 0))],
        out_specs=pl.BlockSpec((1, C), lambda i: (0, 0)),
        out_shape=jax.ShapeDtypeStruct((1, C), jnp.float32),
        scratch_shapes=[pltpu.VMEM((1, 128), jnp.float32)],
    )(p2, nd, b1.reshape(1, H), Wr)

    return out
